# baseline (speedup 1.0000x reference)
import functools

import jax
import jax.numpy as jnp
from jax import lax
from jax.experimental import pallas as pl
from jax.experimental.pallas import tpu as pltpu
from jax.experimental.pallas import tpu_sc as plsc

N_ROWS = 10000
R = 10240
PAD_ROW = 10000
NCORES, NSUB, LANES = 2, 16, 16
K = 128
NCHUNK = 160
NBUF = 4
ROWS_PER_TILE = R // NSUB
DW = 16
DEG_CHUNK = NCHUNK // NCORES
BR = 1024
GRID = R // BR

_MESH = dict(core_axis_name="c", subcore_axis_name="s",
             num_cores=NCORES, num_subcores=NSUB)


def _zero_fill(buf, rows, width):
    zeros16 = jnp.zeros((LANES,), jnp.float32)

    def zrow(i, _):
        for j in range(width // LANES):
            buf[i, pl.ds(j * LANES, LANES)] = zeros16
        return 0

    lax.fori_loop(0, rows, zrow, 0)


def _make_agg(dh, nphase):
    cpp = NCHUNK // nphase
    assert cpp % NBUF == 0
    mesh = plsc.VectorSubcoreMesh(**_MESH)

    @functools.partial(
        pl.kernel,
        out_type=jax.ShapeDtypeStruct((R, 128), jnp.float32),
        mesh=mesh,
        compiler_params=pltpu.CompilerParams(use_tc_tiling_on_sc=False),
        scratch_types=[
            pltpu.VMEM((cpp, K), jnp.int32),
            pltpu.VMEM((cpp, K), jnp.int32),
            pltpu.VMEM((NBUF, K, dh), jnp.float32),
            pltpu.VMEM_SHARED((R, dh), jnp.float32),
            pltpu.VMEM_SHARED((R, dh), jnp.float32),
            pltpu.SemaphoreType.DMA((NBUF,)),
        ],
    )
    def agg(table_hbm, src_hbm, dst_hbm, out_hbm, src_v, dst_v, gbuf,
            tab_sh, acc_sh, gsem):
        c = lax.axis_index("c")
        s = lax.axis_index("s")
        row0 = s * ROWS_PER_TILE
        col0 = c * dh
        pltpu.sync_copy(table_hbm.at[pl.ds(row0, ROWS_PER_TILE), pl.ds(col0, dh)],
                        tab_sh.at[pl.ds(row0, ROWS_PER_TILE)])
        _zero_fill(gbuf.at[0], K, dh)
        for j in range(ROWS_PER_TILE // K):
            pltpu.sync_copy(gbuf.at[0], acc_sh.at[pl.ds(row0 + j * K, K)])
        plsc.subcore_barrier()

        def fire(b, i):
            pltpu.async_copy(tab_sh.at[src_v.at[i]], gbuf.at[b], gsem.at[b])

        def drain(b, i):
            pltpu.make_async_copy(tab_sh.at[src_v.at[i]], gbuf.at[b],
                                  gsem.at[b]).wait()

        def scat(b, i):
            pltpu.sync_copy(gbuf.at[b], acc_sh.at[dst_v.at[i]], add=True)

        for p in range(nphase):
            pltpu.sync_copy(src_hbm.at[s, pl.ds(p * cpp, cpp)], src_v)
            pltpu.sync_copy(dst_hbm.at[s, pl.ds(p * cpp, cpp)], dst_v)
            for b in range(NBUF):
                fire(b, b)

            def body(outer, _):
                base = outer * NBUF
                for b in range(NBUF):
                    i = base + b
                    drain(b, i)
                    scat(b, i)
                    fire(b, i + NBUF)
                return 0

            lax.fori_loop(0, cpp // NBUF - 1, body, 0)
            for b in range(NBUF):
                i = cpp - NBUF + b
                drain(b, i)
                scat(b, i)

        plsc.subcore_barrier()
        pltpu.sync_copy(acc_sh.at[pl.ds(row0, ROWS_PER_TILE)],
                        out_hbm.at[pl.ds(row0, ROWS_PER_TILE), pl.ds(col0, dh)])

    return agg


def _make_deg():
    mesh = plsc.VectorSubcoreMesh(**_MESH)

    @functools.partial(
        pl.kernel,
        out_type=jax.ShapeDtypeStruct((R, 128), jnp.float32),
        mesh=mesh,
        compiler_params=pltpu.CompilerParams(use_tc_tiling_on_sc=False),
        scratch_types=[
            pltpu.VMEM((DEG_CHUNK, K), jnp.int32),
            pltpu.VMEM((K, DW), jnp.float32),
            pltpu.VMEM_SHARED((R, DW), jnp.float32),
        ],
    )
    def deg(dst_hbm, out_hbm, dst_v, obuf, deg_sh):
        c = lax.axis_index("c")
        s = lax.axis_index("s")
        row0 = s * ROWS_PER_TILE
        pltpu.sync_copy(dst_hbm.at[s, pl.ds(c * DEG_CHUNK, DEG_CHUNK)], dst_v)
        _zero_fill(obuf, K, DW)
        for j in range(ROWS_PER_TILE // K):
            pltpu.sync_copy(obuf, deg_sh.at[pl.ds(row0 + j * K, K)])
        ones16 = jnp.ones((LANES,), jnp.float32)

        def orow(i, _):
            obuf[i, pl.ds(0, LANES)] = ones16
            return 0

        lax.fori_loop(0, K, orow, 0)
        plsc.subcore_barrier()

        def body(g, _):
            pltpu.sync_copy(obuf, deg_sh.at[dst_v.at[g]], add=True)
            return 0

        lax.fori_loop(0, DEG_CHUNK, body, 0)
        plsc.subcore_barrier()
        pltpu.sync_copy(deg_sh.at[pl.ds(row0, ROWS_PER_TILE)],
                        out_hbm.at[pl.ds(row0, ROWS_PER_TILE), pl.ds(c * DW, DW)])

    return deg


_agg64 = _make_agg(64, 4)
_agg32 = _make_agg(32, 2)
_deg = _make_deg()

DPACK = BR // 128


def _tc1(deg2, xp, W1):
    def body(deg_r, x_r, w_r, dinv_r, hp_r):
        degv = deg_r[:, 0:1] + deg_r[:, 16:17] + 1.0
        dinv = lax.rsqrt(degv)
        h = jnp.dot(x_r[...], w_r[...], preferred_element_type=jnp.float32) * dinv
        dinv_r[...] = dinv
        hp_r[...] = h

    return pl.pallas_call(
        body,
        grid=(GRID,),
        in_specs=[
            pl.BlockSpec((BR, 128), lambda i: (i, 0)),
            pl.BlockSpec((BR, 128), lambda i: (i, 0)),
            pl.BlockSpec((128, 128), lambda i: (0, 0)),
        ],
        out_specs=[
            pl.BlockSpec((BR, 1), lambda i: (i, 0)),
            pl.BlockSpec((BR, 128), lambda i: (i, 0)),
        ],
        out_shape=[
            jax.ShapeDtypeStruct((R, 1), jnp.float32),
            jax.ShapeDtypeStruct((R, 128), jnp.float32),
        ],
    )(deg2, xp, W1)


def _tc_mid(acc, hp, dinv, b, W, dout):
    def body(acc_r, hp_r, dinv_r, b_r, w_r, o_r):
        dinvv = dinv_r[...]
        comb = acc_r[...] + hp_r[...]
        z = jnp.maximum(comb * dinvv + b_r[...], 0.0)
        h = jnp.dot(z, w_r[...], preferred_element_type=jnp.float32) * dinvv
        if dout < 128:
            h = jnp.concatenate(
                [h, jnp.zeros((BR, 128 - dout), jnp.float32)], axis=-1)
        o_r[...] = h

    return pl.pallas_call(
        body,
        grid=(GRID,),
        in_specs=[
            pl.BlockSpec((BR, 128), lambda i: (i, 0)),
            pl.BlockSpec((BR, 128), lambda i: (i, 0)),
            pl.BlockSpec((BR, 1), lambda i: (i, 0)),
            pl.BlockSpec((1, 128), lambda i: (0, 0)),
            pl.BlockSpec((128, dout), lambda i: (0, 0)),
        ],
        out_specs=pl.BlockSpec((BR, 128), lambda i: (i, 0)),
        out_shape=jax.ShapeDtypeStruct((R, 128), jnp.float32),
    )(acc, hp, dinv, b, W)


def _tc_fin(acc, hp, dinv, b3):
    def body(acc_r, hp_r, dinv_r, b_r, o_r):
        dinvv = dinv_r[...]
        y = (acc_r[:, :64] + hp_r[:, :64]) * dinvv + b_r[...]
        m = jnp.max(y, axis=-1, keepdims=True)
        lse = jnp.log(jnp.sum(jnp.exp(y - m), axis=-1, keepdims=True))
        o_r[...] = y - m - lse

    return pl.pallas_call(
        body,
        grid=(GRID,),
        in_specs=[
            pl.BlockSpec((BR, 128), lambda i: (i, 0)),
            pl.BlockSpec((BR, 128), lambda i: (i, 0)),
            pl.BlockSpec((BR, 1), lambda i: (i, 0)),
            pl.BlockSpec((1, 64), lambda i: (0, 0)),
        ],
        out_specs=pl.BlockSpec((BR, 64), lambda i: (i, 0)),
        out_shape=jax.ShapeDtypeStruct((N_ROWS, 64), jnp.float32),
    )(acc, hp, dinv, b3)


@jax.jit
def kernel(x, edge_index, W1, b1, W2, b2, W3, b3):
    ei = edge_index.astype(jnp.int32)
    e = ei.shape[1]
    eip = jnp.pad(ei, ((0, 0), (0, NSUB * NCHUNK * K - e)),
                  constant_values=PAD_ROW).reshape(2, NSUB, NCHUNK, K)
    srcp, dstp = eip[0], eip[1]
    xp = jnp.pad(x, ((0, R - x.shape[0]), (0, 0)))

    deg2 = _deg(dstp)
    dinv, hp1 = _tc1(deg2, xp, W1)
    acc1 = _agg64(hp1, srcp, dstp)
    hp2 = _tc_mid(acc1, hp1, dinv, b1.reshape(1, 128), W2, 128)
    acc2 = _agg64(hp2, srcp, dstp)
    hp3 = _tc_mid(acc2, hp2, dinv, b2.reshape(1, 128), W3, 64)
    acc3 = _agg32(hp3, srcp, dstp)
    return _tc_fin(acc3, hp3, dinv, b3.reshape(1, 64))


# speedup vs baseline: 1.0312x; 1.0312x over previous
import functools

import jax
import jax.numpy as jnp
from jax import lax
from jax.experimental import pallas as pl
from jax.experimental.pallas import tpu as pltpu
from jax.experimental.pallas import tpu_sc as plsc

N_ROWS = 10000
R = 10240
PAD_ROW = 10000
NCORES, NSUB, LANES = 2, 16, 16
K = 128
NCHUNK = 160
NBUF = 2
ROWS_PER_TILE = R // NSUB
DW = 16
DEG_CHUNK = NCHUNK // NCORES
BR = 1024
GRID = R // BR

_MESH = dict(core_axis_name="c", subcore_axis_name="s",
             num_cores=NCORES, num_subcores=NSUB)


def _zero_fill(buf, rows, width):
    zeros16 = jnp.zeros((LANES,), jnp.float32)

    def zrow(i, _):
        for j in range(width // LANES):
            buf[i, pl.ds(j * LANES, LANES)] = zeros16
        return 0

    lax.fori_loop(0, rows, zrow, 0)


def _make_agg(dh, nphase):
    cpp = NCHUNK // nphase
    assert cpp % NBUF == 0
    mesh = plsc.VectorSubcoreMesh(**_MESH)

    @functools.partial(
        pl.kernel,
        out_type=jax.ShapeDtypeStruct((R, 128), jnp.float32),
        mesh=mesh,
        compiler_params=pltpu.CompilerParams(use_tc_tiling_on_sc=False),
        scratch_types=[
            pltpu.VMEM((cpp, K), jnp.int32),
            pltpu.VMEM((cpp, K), jnp.int32),
            pltpu.VMEM((NBUF, K, dh), jnp.float32),
            pltpu.VMEM_SHARED((R, dh), jnp.float32),
            pltpu.VMEM_SHARED((R, dh), jnp.float32),
            pltpu.SemaphoreType.DMA((NBUF,)),
        ],
    )
    def agg(table_hbm, src_hbm, dst_hbm, out_hbm, src_v, dst_v, gbuf,
            tab_sh, acc_sh, gsem):
        c = lax.axis_index("c")
        s = lax.axis_index("s")
        row0 = s * ROWS_PER_TILE
        col0 = c * dh
        pltpu.sync_copy(table_hbm.at[pl.ds(row0, ROWS_PER_TILE), pl.ds(col0, dh)],
                        tab_sh.at[pl.ds(row0, ROWS_PER_TILE)])
        _zero_fill(gbuf.at[0], K, dh)
        for j in range(ROWS_PER_TILE // K):
            pltpu.sync_copy(gbuf.at[0], acc_sh.at[pl.ds(row0 + j * K, K)])
        plsc.subcore_barrier()

        def fire(b, i):
            pltpu.async_copy(tab_sh.at[src_v.at[i]], gbuf.at[b], gsem.at[b])

        def drain(b, i):
            pltpu.make_async_copy(tab_sh.at[src_v.at[i]], gbuf.at[b],
                                  gsem.at[b]).wait()

        def scat(b, i):
            pltpu.sync_copy(gbuf.at[b], acc_sh.at[dst_v.at[i]], add=True)

        for p in range(nphase):
            pltpu.sync_copy(src_hbm.at[s, pl.ds(p * cpp, cpp)], src_v)
            pltpu.sync_copy(dst_hbm.at[s, pl.ds(p * cpp, cpp)], dst_v)
            for b in range(NBUF):
                fire(b, b)

            def body(outer, _):
                base = outer * NBUF
                for b in range(NBUF):
                    i = base + b
                    drain(b, i)
                    scat(b, i)
                    fire(b, i + NBUF)
                return 0

            lax.fori_loop(0, cpp // NBUF - 1, body, 0)
            for b in range(NBUF):
                i = cpp - NBUF + b
                drain(b, i)
                scat(b, i)

        plsc.subcore_barrier()
        pltpu.sync_copy(acc_sh.at[pl.ds(row0, ROWS_PER_TILE)],
                        out_hbm.at[pl.ds(row0, ROWS_PER_TILE), pl.ds(col0, dh)])

    return agg


def _make_deg():
    mesh = plsc.VectorSubcoreMesh(**_MESH)

    @functools.partial(
        pl.kernel,
        out_type=jax.ShapeDtypeStruct((R, 128), jnp.float32),
        mesh=mesh,
        compiler_params=pltpu.CompilerParams(use_tc_tiling_on_sc=False),
        scratch_types=[
            pltpu.VMEM((DEG_CHUNK, K), jnp.int32),
            pltpu.VMEM((K, DW), jnp.float32),
            pltpu.VMEM_SHARED((R, DW), jnp.float32),
        ],
    )
    def deg(dst_hbm, out_hbm, dst_v, obuf, deg_sh):
        c = lax.axis_index("c")
        s = lax.axis_index("s")
        row0 = s * ROWS_PER_TILE
        pltpu.sync_copy(dst_hbm.at[s, pl.ds(c * DEG_CHUNK, DEG_CHUNK)], dst_v)
        _zero_fill(obuf, K, DW)
        for j in range(ROWS_PER_TILE // K):
            pltpu.sync_copy(obuf, deg_sh.at[pl.ds(row0 + j * K, K)])
        ones16 = jnp.ones((LANES,), jnp.float32)

        def orow(i, _):
            obuf[i, pl.ds(0, LANES)] = ones16
            return 0

        lax.fori_loop(0, K, orow, 0)
        plsc.subcore_barrier()

        def body(g, _):
            pltpu.sync_copy(obuf, deg_sh.at[dst_v.at[g]], add=True)
            return 0

        lax.fori_loop(0, DEG_CHUNK, body, 0)
        plsc.subcore_barrier()
        pltpu.sync_copy(deg_sh.at[pl.ds(row0, ROWS_PER_TILE)],
                        out_hbm.at[pl.ds(row0, ROWS_PER_TILE), pl.ds(c * DW, DW)])

    return deg


_agg64 = _make_agg(64, 2)
_agg32 = _make_agg(32, 2)
_deg = _make_deg()


def _tc1(deg2, xp, W1):
    def body(deg_r, x_r, w_r, dinv_r, hp_r):
        degv = deg_r[:, 0:1] + deg_r[:, 16:17] + 1.0
        dinv = lax.rsqrt(degv)
        h = jnp.dot(x_r[...], w_r[...], preferred_element_type=jnp.float32) * dinv
        dinv_r[...] = dinv
        hp_r[...] = h

    return pl.pallas_call(
        body,
        grid=(GRID,),
        in_specs=[
            pl.BlockSpec((BR, 128), lambda i: (i, 0)),
            pl.BlockSpec((BR, 128), lambda i: (i, 0)),
            pl.BlockSpec((128, 128), lambda i: (0, 0)),
        ],
        out_specs=[
            pl.BlockSpec((BR, 1), lambda i: (i, 0)),
            pl.BlockSpec((BR, 128), lambda i: (i, 0)),
        ],
        out_shape=[
            jax.ShapeDtypeStruct((R, 1), jnp.float32),
            jax.ShapeDtypeStruct((R, 128), jnp.float32),
        ],
    )(deg2, xp, W1)


def _tc_mid(acc, hp, dinv, b, W, dout):
    def body(acc_r, hp_r, dinv_r, b_r, w_r, o_r):
        dinvv = dinv_r[...]
        comb = acc_r[...] + hp_r[...]
        z = jnp.maximum(comb * dinvv + b_r[...], 0.0)
        h = jnp.dot(z, w_r[...], preferred_element_type=jnp.float32) * dinvv
        if dout < 128:
            h = jnp.concatenate(
                [h, jnp.zeros((BR, 128 - dout), jnp.float32)], axis=-1)
        o_r[...] = h

    return pl.pallas_call(
        body,
        grid=(GRID,),
        in_specs=[
            pl.BlockSpec((BR, 128), lambda i: (i, 0)),
            pl.BlockSpec((BR, 128), lambda i: (i, 0)),
            pl.BlockSpec((BR, 1), lambda i: (i, 0)),
            pl.BlockSpec((1, 128), lambda i: (0, 0)),
            pl.BlockSpec((128, dout), lambda i: (0, 0)),
        ],
        out_specs=pl.BlockSpec((BR, 128), lambda i: (i, 0)),
        out_shape=jax.ShapeDtypeStruct((R, 128), jnp.float32),
    )(acc, hp, dinv, b, W)


def _tc_fin(acc, hp, dinv, b3):
    def body(acc_r, hp_r, dinv_r, b_r, o_r):
        dinvv = dinv_r[...]
        y = (acc_r[:, :64] + hp_r[:, :64]) * dinvv + b_r[...]
        m = jnp.max(y, axis=-1, keepdims=True)
        lse = jnp.log(jnp.sum(jnp.exp(y - m), axis=-1, keepdims=True))
        o_r[...] = y - m - lse

    return pl.pallas_call(
        body,
        grid=(GRID,),
        in_specs=[
            pl.BlockSpec((BR, 128), lambda i: (i, 0)),
            pl.BlockSpec((BR, 128), lambda i: (i, 0)),
            pl.BlockSpec((BR, 1), lambda i: (i, 0)),
            pl.BlockSpec((1, 64), lambda i: (0, 0)),
        ],
        out_specs=pl.BlockSpec((BR, 64), lambda i: (i, 0)),
        out_shape=jax.ShapeDtypeStruct((N_ROWS, 64), jnp.float32),
    )(acc, hp, dinv, b3)


@jax.jit
def kernel(x, edge_index, W1, b1, W2, b2, W3, b3):
    ei = edge_index.astype(jnp.int32)
    e = ei.shape[1]
    eip = jnp.pad(ei, ((0, 0), (0, NSUB * NCHUNK * K - e)),
                  constant_values=PAD_ROW).reshape(2, NSUB, NCHUNK, K)
    srcp, dstp = eip[0], eip[1]
    xp = jnp.pad(x, ((0, R - x.shape[0]), (0, 0)))

    deg2 = _deg(dstp)
    dinv, hp1 = _tc1(deg2, xp, W1)
    acc1 = _agg64(hp1, srcp, dstp)
    hp2 = _tc_mid(acc1, hp1, dinv, b1.reshape(1, 128), W2, 128)
    acc2 = _agg64(hp2, srcp, dstp)
    hp3 = _tc_mid(acc2, hp2, dinv, b2.reshape(1, 128), W3, 64)
    acc3 = _agg32(hp3, srcp, dstp)
    return _tc_fin(acc3, hp3, dinv, b3.reshape(1, 64))


# speedup vs baseline: 1.0323x; 1.0011x over previous
import functools

import jax
import jax.numpy as jnp
from jax import lax
from jax.experimental import pallas as pl
from jax.experimental.pallas import tpu as pltpu
from jax.experimental.pallas import tpu_sc as plsc

N_ROWS = 10000
R = 10240
PAD_ROW = 10000
NCORES, NSUB, LANES = 2, 16, 16
K = 128
NCHUNK = 160
NBUF = 2
ROWS_PER_TILE = R // NSUB
DW = 16
DEG_CHUNK = NCHUNK // NCORES
BR = 1024
GRID = R // BR

_MESH = dict(core_axis_name="c", subcore_axis_name="s",
             num_cores=NCORES, num_subcores=NSUB)


def _zero_fill(buf, rows, width):
    zeros16 = jnp.zeros((LANES,), jnp.float32)

    def zrow(i, _):
        for j in range(width // LANES):
            buf[i, pl.ds(j * LANES, LANES)] = zeros16
        return 0

    lax.fori_loop(0, rows, zrow, 0)


def _make_agg(dh, nphase):
    cpp = NCHUNK // nphase
    assert cpp % NBUF == 0
    mesh = plsc.VectorSubcoreMesh(**_MESH)

    @functools.partial(
        pl.kernel,
        out_type=jax.ShapeDtypeStruct((R, 128), jnp.float32),
        mesh=mesh,
        compiler_params=pltpu.CompilerParams(use_tc_tiling_on_sc=False),
        scratch_types=[
            pltpu.VMEM((cpp, K), jnp.int32),
            pltpu.VMEM((cpp, K), jnp.int32),
            pltpu.VMEM((NBUF, K, dh), jnp.float32),
            pltpu.VMEM_SHARED((R, dh), jnp.float32),
            pltpu.VMEM_SHARED((R, dh), jnp.float32),
            pltpu.SemaphoreType.DMA((NBUF,)),
        ],
    )
    def agg(table_hbm, src_hbm, dst_hbm, out_hbm, src_v, dst_v, gbuf,
            tab_sh, acc_sh, gsem):
        c = lax.axis_index("c")
        s = lax.axis_index("s")
        row0 = s * ROWS_PER_TILE
        col0 = c * dh
        pltpu.sync_copy(table_hbm.at[pl.ds(row0, ROWS_PER_TILE), pl.ds(col0, dh)],
                        tab_sh.at[pl.ds(row0, ROWS_PER_TILE)])
        _zero_fill(gbuf.at[0], K, dh)
        for j in range(ROWS_PER_TILE // K):
            pltpu.sync_copy(gbuf.at[0], acc_sh.at[pl.ds(row0 + j * K, K)])
        plsc.subcore_barrier()

        def fire(b, i):
            pltpu.async_copy(tab_sh.at[src_v.at[i]], gbuf.at[b], gsem.at[b])

        def drain(b, i):
            pltpu.make_async_copy(tab_sh.at[src_v.at[i]], gbuf.at[b],
                                  gsem.at[b]).wait()

        def scat(b, i):
            pltpu.sync_copy(gbuf.at[b], acc_sh.at[dst_v.at[i]], add=True)

        for p in range(nphase):
            pltpu.sync_copy(src_hbm.at[s, pl.ds(p * cpp, cpp)], src_v)
            pltpu.sync_copy(dst_hbm.at[s, pl.ds(p * cpp, cpp)], dst_v)
            for b in range(NBUF):
                fire(b, b)

            def body(outer, _):
                base = outer * NBUF
                for b in range(NBUF):
                    i = base + b
                    drain(b, i)
                    scat(b, i)
                    fire(b, i + NBUF)
                return 0

            lax.fori_loop(0, cpp // NBUF - 1, body, 0)
            for b in range(NBUF):
                i = cpp - NBUF + b
                drain(b, i)
                scat(b, i)

        plsc.subcore_barrier()
        pltpu.sync_copy(acc_sh.at[pl.ds(row0, ROWS_PER_TILE)],
                        out_hbm.at[pl.ds(row0, ROWS_PER_TILE), pl.ds(col0, dh)])

    return agg


def _make_deg():
    mesh = plsc.VectorSubcoreMesh(**_MESH)

    @functools.partial(
        pl.kernel,
        out_type=jax.ShapeDtypeStruct((R, 128), jnp.float32),
        mesh=mesh,
        compiler_params=pltpu.CompilerParams(use_tc_tiling_on_sc=False),
        scratch_types=[
            pltpu.VMEM((DEG_CHUNK, K), jnp.int32),
            pltpu.VMEM((K, DW), jnp.float32),
            pltpu.VMEM_SHARED((R, DW), jnp.float32),
        ],
    )
    def deg(dst_hbm, out_hbm, dst_v, obuf, deg_sh):
        c = lax.axis_index("c")
        s = lax.axis_index("s")
        row0 = s * ROWS_PER_TILE
        pltpu.sync_copy(dst_hbm.at[s, pl.ds(c * DEG_CHUNK, DEG_CHUNK)], dst_v)
        _zero_fill(obuf, K, DW)
        for j in range(ROWS_PER_TILE // K):
            pltpu.sync_copy(obuf, deg_sh.at[pl.ds(row0 + j * K, K)])
        ones16 = jnp.ones((LANES,), jnp.float32)

        def orow(i, _):
            obuf[i, pl.ds(0, LANES)] = ones16
            return 0

        lax.fori_loop(0, K, orow, 0)
        plsc.subcore_barrier()

        def body(g, _):
            pltpu.sync_copy(obuf, deg_sh.at[dst_v.at[g]], add=True)
            return 0

        lax.fori_loop(0, DEG_CHUNK, body, 0)
        plsc.subcore_barrier()
        pltpu.sync_copy(deg_sh.at[pl.ds(row0, ROWS_PER_TILE)],
                        out_hbm.at[pl.ds(row0, ROWS_PER_TILE), pl.ds(c * DW, DW)])

    return deg


_agg64 = _make_agg(64, 2)
_agg32 = _make_agg(32, 1)
_deg = _make_deg()


def _tc1(deg2, xp, W1):
    def body(deg_r, x_r, w_r, dinv_r, hp_r):
        degv = deg_r[:, 0:1] + deg_r[:, 16:17] + 1.0
        dinv = lax.rsqrt(degv)
        h = jnp.dot(x_r[...], w_r[...], preferred_element_type=jnp.float32) * dinv
        dinv_r[...] = dinv
        hp_r[...] = h

    return pl.pallas_call(
        body,
        grid=(GRID,),
        in_specs=[
            pl.BlockSpec((BR, 128), lambda i: (i, 0)),
            pl.BlockSpec((BR, 128), lambda i: (i, 0)),
            pl.BlockSpec((128, 128), lambda i: (0, 0)),
        ],
        out_specs=[
            pl.BlockSpec((BR, 1), lambda i: (i, 0)),
            pl.BlockSpec((BR, 128), lambda i: (i, 0)),
        ],
        out_shape=[
            jax.ShapeDtypeStruct((R, 1), jnp.float32),
            jax.ShapeDtypeStruct((R, 128), jnp.float32),
        ],
    )(deg2, xp, W1)


def _tc_mid(acc, hp, dinv, b, W, dout):
    def body(acc_r, hp_r, dinv_r, b_r, w_r, o_r):
        dinvv = dinv_r[...]
        comb = acc_r[...] + hp_r[...]
        z = jnp.maximum(comb * dinvv + b_r[...], 0.0)
        h = jnp.dot(z, w_r[...], preferred_element_type=jnp.float32) * dinvv
        if dout < 128:
            h = jnp.concatenate(
                [h, jnp.zeros((BR, 128 - dout), jnp.float32)], axis=-1)
        o_r[...] = h

    return pl.pallas_call(
        body,
        grid=(GRID,),
        in_specs=[
            pl.BlockSpec((BR, 128), lambda i: (i, 0)),
            pl.BlockSpec((BR, 128), lambda i: (i, 0)),
            pl.BlockSpec((BR, 1), lambda i: (i, 0)),
            pl.BlockSpec((1, 128), lambda i: (0, 0)),
            pl.BlockSpec((128, dout), lambda i: (0, 0)),
        ],
        out_specs=pl.BlockSpec((BR, 128), lambda i: (i, 0)),
        out_shape=jax.ShapeDtypeStruct((R, 128), jnp.float32),
    )(acc, hp, dinv, b, W)


def _tc_fin(acc, hp, dinv, b3):
    def body(acc_r, hp_r, dinv_r, b_r, o_r):
        dinvv = dinv_r[...]
        y = (acc_r[:, :64] + hp_r[:, :64]) * dinvv + b_r[...]
        m = jnp.max(y, axis=-1, keepdims=True)
        lse = jnp.log(jnp.sum(jnp.exp(y - m), axis=-1, keepdims=True))
        o_r[...] = y - m - lse

    return pl.pallas_call(
        body,
        grid=(GRID,),
        in_specs=[
            pl.BlockSpec((BR, 128), lambda i: (i, 0)),
            pl.BlockSpec((BR, 128), lambda i: (i, 0)),
            pl.BlockSpec((BR, 1), lambda i: (i, 0)),
            pl.BlockSpec((1, 64), lambda i: (0, 0)),
        ],
        out_specs=pl.BlockSpec((BR, 64), lambda i: (i, 0)),
        out_shape=jax.ShapeDtypeStruct((N_ROWS, 64), jnp.float32),
    )(acc, hp, dinv, b3)


@jax.jit
def kernel(x, edge_index, W1, b1, W2, b2, W3, b3):
    ei = edge_index.astype(jnp.int32)
    e = ei.shape[1]
    eip = jnp.pad(ei, ((0, 0), (0, NSUB * NCHUNK * K - e)),
                  constant_values=PAD_ROW).reshape(2, NSUB, NCHUNK, K)
    srcp, dstp = eip[0], eip[1]
    xp = jnp.pad(x, ((0, R - x.shape[0]), (0, 0)))

    deg2 = _deg(dstp)
    dinv, hp1 = _tc1(deg2, xp, W1)
    acc1 = _agg64(hp1, srcp, dstp)
    hp2 = _tc_mid(acc1, hp1, dinv, b1.reshape(1, 128), W2, 128)
    acc2 = _agg64(hp2, srcp, dstp)
    hp3 = _tc_mid(acc2, hp2, dinv, b2.reshape(1, 128), W3, 64)
    acc3 = _agg32(hp3, srcp, dstp)
    return _tc_fin(acc3, hp3, dinv, b3.reshape(1, 64))
